# baseline (speedup 1.0000x reference)
import functools

import jax
import jax.numpy as jnp
from jax import lax
from jax.experimental import pallas as pl
from jax.experimental.pallas import tpu as pltpu
from jax.experimental.pallas import tpu_sc as plsc

_TOPK = 8
_CHUNK = 64
_NL = 16


def _reduce_kernel(s_ref, t_ref, g_ref, a1_ref, a2_ref, gm_ref):
    j = pl.program_id(0)
    R = s_ref.shape[1]
    B = s_ref.shape[2]
    nsub = _CHUNK // 8

    def body(it, carry):
        a1, a2, gm = carry
        base = pl.multiple_of(it * _CHUNK, _CHUNK)
        s = s_ref[0, pl.ds(base, _CHUNK), :]
        t = t_ref[0, pl.ds(base, _CHUNK), :]
        g = g_ref[0, pl.ds(base, _CHUNK), :]
        d1 = s - g
        d2 = s - t
        a1 = a1 + jnp.sum((d1 * d1).reshape(nsub, 8, B), axis=0)
        a2 = a2 + jnp.sum((d2 * d2).reshape(nsub, 8, B), axis=0)
        gm = jnp.maximum(gm, jnp.max(g.reshape(nsub, 8, B), axis=0))
        return a1, a2, gm

    zero = jnp.zeros((8, B), jnp.float32)
    ninf = jnp.full((8, B), -jnp.inf, jnp.float32)
    a1, a2, gm = jax.lax.fori_loop(0, R // _CHUNK, body, (zero, zero, ninf))
    a1_ref[pl.ds(j, 1), :] = jnp.sum(a1, axis=0, keepdims=True)
    a2_ref[pl.ds(j, 1), :] = jnp.sum(a2, axis=0, keepdims=True)
    gm_ref[pl.ds(j, 1), :] = jnp.max(gm, axis=0, keepdims=True)


def _sc_shuffle(x, idx):
    return lax.gather(
        x, idx[:, None],
        lax.GatherDimensionNumbers(offset_dims=(), collapsed_slice_dims=(0,),
                                   start_index_map=(0,)),
        (1,), mode=lax.GatherScatterMode.PROMISE_IN_BOUNDS)


def _sc_allsum(x, iota):
    for k in (1, 2, 4, 8):
        x = x + _sc_shuffle(x, iota ^ k)
    return x


def _sc_allmax(x, iota):
    for k in (1, 2, 4, 8):
        x = jnp.maximum(x, _sc_shuffle(x, iota ^ k))
    return x


def _sc_mining_kernel(a1_hbm, a2_hbm, gm_hbm, tw_hbm, out_hbm,
                      a1_v, a2_v, gm_v, tw_v, out_v, *, nj, nb, hw):
    cid = lax.axis_index("c")
    sid = lax.axis_index("s")
    nc = nb // _NL

    @pl.when((cid == 0) & (sid == 0))
    def _():
        pltpu.sync_copy(a1_hbm, a1_v)
        pltpu.sync_copy(a2_hbm, a2_v)
        pltpu.sync_copy(gm_hbm, gm_v)
        pltpu.sync_copy(tw_hbm, tw_v)
        iota = lax.iota(jnp.int32, _NL)
        notc = []
        for j in range(nj):
            gv = gm_v[0, j, :]
            for c in range(1, nc):
                gv = jnp.maximum(gv, gm_v[c, j, :])
            gj = _sc_allmax(gv, iota)
            notc.append(jnp.where(gj == 1.0, 0.0, 1.0))
        zf = jnp.zeros((_NL,), jnp.float32)

        def chunk_body(c, carry):
            mse_acc, ohkm_acc = carry
            vals = []
            for j in range(nj):
                twv = tw_v[c, j, :]
                w = (twv * twv) * (a1_v[c, j, :] + notc[j] * a2_v[c, j, :])
                mse_acc = mse_acc + w
                vals.append((0.5 / hw) * w)
            negbig = jnp.float32(-3.4e38)
            one = jnp.ones((_NL,), jnp.float32)
            excl = [zf for _ in range(nj)]
            acc = zf
            for _ in range(_TOPK):
                m = vals[0] + excl[0] * negbig
                for j in range(1, nj):
                    m = jnp.maximum(m, vals[j] + excl[j] * negbig)
                acc = acc + m
                taken = zf
                for j in range(nj):
                    hit = ((one - excl[j]) * (one - taken)
                           * jnp.where(vals[j] == m, one, zf))
                    excl[j] = excl[j] + hit
                    taken = taken + hit
            return mse_acc, ohkm_acc + acc

        mse_acc, ohkm_acc = jax.lax.fori_loop(0, nc, chunk_body, (zf, zf))
        mse = _sc_allsum(mse_acc, iota) / (nb * hw)
        ohkm = _sc_allsum(ohkm_acc, iota) / (_TOPK * nb)
        r = jnp.where(iota == 0, ohkm,
                      jnp.where(iota == 1, mse / nj,
                                jnp.where(iota == 2, ohkm + mse, 0.0)))
        out_v[...] = r
        pltpu.sync_copy(out_v, out_hbm)


def kernel(output_s, output_t, target, target_weight):
    B, J, H, W = output_s.shape
    HW = H * W
    st = jnp.transpose(output_s, (1, 2, 3, 0)).reshape(J, HW, B)
    tt = jnp.transpose(output_t, (1, 2, 3, 0)).reshape(J, HW, B)
    gt = jnp.transpose(target, (1, 2, 3, 0)).reshape(J, HW, B)
    mat = jax.ShapeDtypeStruct((J, B), jnp.float32)
    a1, a2, gm = pl.pallas_call(
        _reduce_kernel,
        grid=(J,),
        in_specs=[
            pl.BlockSpec((1, HW, B), lambda j: (j, 0, 0)),
            pl.BlockSpec((1, HW, B), lambda j: (j, 0, 0)),
            pl.BlockSpec((1, HW, B), lambda j: (j, 0, 0)),
        ],
        out_specs=[
            pl.BlockSpec((J, B), lambda j: (0, 0)),
            pl.BlockSpec((J, B), lambda j: (0, 0)),
            pl.BlockSpec((J, B), lambda j: (0, 0)),
        ],
        out_shape=[mat, mat, mat],
    )(st, tt, gt)

    nc = B // _NL
    def chunked(x):
        return jnp.transpose(x.reshape(J, nc, _NL), (1, 0, 2))
    twc = chunked(jnp.transpose(target_weight.reshape(B, J)))
    mesh = plsc.VectorSubcoreMesh(core_axis_name="c", subcore_axis_name="s")
    cvec = pltpu.VMEM((nc, J, _NL), jnp.float32)
    out = pl.kernel(
        functools.partial(_sc_mining_kernel, nj=J, nb=B, hw=float(HW)),
        out_type=jax.ShapeDtypeStruct((_NL,), jnp.float32),
        mesh=mesh,
        scratch_types=[cvec, cvec, cvec, cvec,
                       pltpu.VMEM((_NL,), jnp.float32)],
    )(chunked(a1), chunked(a2), chunked(gm), twc)
    return (out[0], out[1], out[2])


# speedup vs baseline: 1.1384x; 1.1384x over previous
import functools

import jax
import jax.numpy as jnp
from jax import lax
from jax.experimental import pallas as pl
from jax.experimental.pallas import tpu as pltpu
from jax.experimental.pallas import tpu_sc as plsc

_TOPK = 8
_CHUNK = 64
_NL = 16
_NW = 8


def _reduce_kernel(tw_ref, s_ref, t_ref, g_ref, w_ref, mse_ref,
                   a1_ref, a2_ref, gm_ref, *, nj):
    j = pl.program_id(0)
    R = s_ref.shape[1]
    B = s_ref.shape[2]
    nsub = _CHUNK // 8

    def body(it, carry):
        a1, a2, gm = carry
        base = pl.multiple_of(it * _CHUNK, _CHUNK)
        s = s_ref[0, pl.ds(base, _CHUNK), :]
        t = t_ref[0, pl.ds(base, _CHUNK), :]
        g = g_ref[0, pl.ds(base, _CHUNK), :]
        d1 = s - g
        d2 = s - t
        a1 = a1 + jnp.sum((d1 * d1).reshape(nsub, 8, B), axis=0)
        a2 = a2 + jnp.sum((d2 * d2).reshape(nsub, 8, B), axis=0)
        gm = jnp.maximum(gm, jnp.max(g.reshape(nsub, 8, B), axis=0))
        return a1, a2, gm

    zero = jnp.zeros((8, B), jnp.float32)
    ninf = jnp.full((8, B), -jnp.inf, jnp.float32)
    a1, a2, gm = jax.lax.fori_loop(0, R // _CHUNK, body, (zero, zero, ninf))
    a1_ref[pl.ds(j, 1), :] = jnp.sum(a1, axis=0, keepdims=True)
    a2_ref[pl.ds(j, 1), :] = jnp.sum(a2, axis=0, keepdims=True)
    gm_ref[pl.ds(j, 1), :] = jnp.max(gm, axis=0, keepdims=True)

    @pl.when(j == nj - 1)
    def _weights():
        J, B2 = a1_ref.shape
        tw = tw_ref[...]
        tw2 = tw * tw
        gmax = jnp.max(gm_ref[...], axis=1, keepdims=True)
        notc = jnp.where(gmax == 1.0, 0.0, 1.0)
        wl = tw2 * (a1_ref[...] + notc * a2_ref[...])
        w_ref[...] = wl
        mse_ref[0, 0] = jnp.sum(wl) / (B2 * R)


def _sc_shuffle(x, idx):
    return lax.gather(
        x, idx[:, None],
        lax.GatherDimensionNumbers(offset_dims=(), collapsed_slice_dims=(0,),
                                   start_index_map=(0,)),
        (1,), mode=lax.GatherScatterMode.PROMISE_IN_BOUNDS)


def _sc_allsum(x, iota):
    for k in (1, 2, 4, 8):
        x = x + _sc_shuffle(x, iota ^ k)
    return x


def _sc_mining_kernel(w_hbm, out_hbm, w_v, part_v, shared_v, all_v,
                      out_v, *, nj, nb, hw):
    cid = lax.axis_index("c")
    sid = lax.axis_index("s")

    @pl.when((cid == 0) & (sid < _NW))
    def _mine():
        pltpu.sync_copy(w_hbm.at[sid], w_v)
        zf = jnp.zeros((_NL,), jnp.float32)
        one = jnp.ones((_NL,), jnp.float32)
        negbig = jnp.float32(-3.4e38)
        vals = [(0.5 / hw) * w_v[j, :] for j in range(nj)]
        excl = [zf for _ in range(nj)]
        acc = zf
        for _ in range(_TOPK):
            m = vals[0] + excl[0] * negbig
            for j in range(1, nj):
                m = jnp.maximum(m, vals[j] + excl[j] * negbig)
            acc = acc + m
            taken = zf
            for j in range(nj):
                hit = ((one - excl[j]) * (one - taken)
                       * jnp.where(vals[j] == m, one, zf))
                excl[j] = excl[j] + hit
                taken = taken + hit
        part_v[0, :] = acc
        pltpu.sync_copy(part_v, shared_v.at[sid])

    plsc.subcore_barrier()

    @pl.when((cid == 0) & (sid == 0))
    def _combine():
        pltpu.sync_copy(shared_v, all_v)
        acc = all_v[0, 0, :]
        for i in range(1, _NW):
            acc = acc + all_v[i, 0, :]
        iota = lax.iota(jnp.int32, _NL)
        ohkm = _sc_allsum(acc, iota) / (_TOPK * nb)
        out_v[...] = jnp.where(iota < 3, ohkm, 0.0)
        pltpu.sync_copy(out_v, out_hbm)


def kernel(output_s, output_t, target, target_weight):
    B, J, H, W = output_s.shape
    HW = H * W
    st = jnp.transpose(output_s, (1, 2, 3, 0)).reshape(J, HW, B)
    tt = jnp.transpose(output_t, (1, 2, 3, 0)).reshape(J, HW, B)
    gt = jnp.transpose(target, (1, 2, 3, 0)).reshape(J, HW, B)
    twt = jnp.transpose(target_weight.reshape(B, J))
    wmat = jax.ShapeDtypeStruct((J, B), jnp.float32)
    scal = jax.ShapeDtypeStruct((1, 1), jnp.float32)
    wl, mse = pl.pallas_call(
        functools.partial(_reduce_kernel, nj=J),
        grid=(J,),
        in_specs=[
            pl.BlockSpec((J, B), lambda j: (0, 0)),
            pl.BlockSpec((1, HW, B), lambda j: (j, 0, 0)),
            pl.BlockSpec((1, HW, B), lambda j: (j, 0, 0)),
            pl.BlockSpec((1, HW, B), lambda j: (j, 0, 0)),
        ],
        out_specs=[pl.BlockSpec((J, B), lambda j: (0, 0)),
                   pl.BlockSpec(memory_space=pltpu.SMEM)],
        out_shape=[wmat, scal],
        scratch_shapes=[
            pltpu.VMEM((J, B), jnp.float32),
            pltpu.VMEM((J, B), jnp.float32),
            pltpu.VMEM((J, B), jnp.float32),
        ],
    )(twt, st, tt, gt)

    wlc = jnp.transpose(wl.reshape(J, _NW, _NL), (1, 0, 2))
    mesh = plsc.VectorSubcoreMesh(core_axis_name="c", subcore_axis_name="s")
    ohkm_vec = pl.kernel(
        functools.partial(_sc_mining_kernel, nj=J, nb=B, hw=float(HW)),
        out_type=jax.ShapeDtypeStruct((_NL,), jnp.float32),
        mesh=mesh,
        scratch_types=[
            pltpu.VMEM((J, _NL), jnp.float32),
            pltpu.VMEM((1, _NL), jnp.float32),
            pltpu.VMEM_SHARED((_NW, 1, _NL), jnp.float32),
            pltpu.VMEM((_NW, 1, _NL), jnp.float32),
            pltpu.VMEM((_NL,), jnp.float32),
        ],
    )(wlc)
    ohkm = ohkm_vec[0]
    mse_s = mse[0, 0]
    return (ohkm, mse_s / J, ohkm + mse_s)
